# Initial kernel scaffold; baseline (speedup 1.0000x reference)
import jax
import jax.numpy as jnp
from jax.experimental import pallas as pl


def kernel(node_features, edge_features, adjacency_mask, node_norm_g, node_norm_b, edge_norm_g, edge_norm_b, Wq, bq, Wk, bk, Wv, bv, Wo, bo, We, be, ff_norm_g, ff_norm_b, W1, b1, W2, b2):
    raise NotImplementedError("write your pallas kernel here")


# speedup vs baseline: 1.3734x; 1.3734x over previous
import jax
import jax.numpy as jnp
from jax.experimental import pallas as pl
from jax.experimental.pallas import tpu as pltpu

_B, _N, _D, _H = 2, 512, 256, 8
_HD = _D // _H
_DFF = 4 * _D
_EPS = 1e-5
_BI = 16
_NEG = -1e30


def _qkv_kernel(x_ref, g_ref, b_ref, wq_ref, bq_ref, wk_ref, wv_ref, bv_ref,
                q_ref, k_ref, v_ref):
    x = x_ref[0]
    m = jnp.mean(x, axis=-1, keepdims=True)
    var = jnp.mean(x * x, axis=-1, keepdims=True) - m * m
    nn = (x - m) * jax.lax.rsqrt(var + _EPS) * g_ref[...] + b_ref[...]
    scale = 1.0 / (_HD ** 0.5)
    q_ref[0] = (jnp.dot(nn, wq_ref[...], preferred_element_type=jnp.float32)
                + bq_ref[...]) * scale
    k_ref[0] = jnp.dot(nn, wk_ref[...], preferred_element_type=jnp.float32)
    v_ref[0] = (jnp.dot(nn, wv_ref[...], preferred_element_type=jnp.float32)
                + bv_ref[...])


def _attn_ffn_kernel(e_ref, msk_ref, q_ref, k_ref, v_ref, x_ref,
                     wegT_ref, sweg_ref, wo_ref, bo_ref, fg_ref, fb_ref,
                     w1_ref, b1_ref, w2_ref, b2_ref, o_ref):
    e = e_ref[0]
    me = jnp.mean(e, axis=-1)
    ve = jnp.mean(e * e, axis=-1) - me * me
    re = jax.lax.rsqrt(ve + _EPS)
    e2 = e.reshape(_BI * _N, _D)
    rawT = jax.lax.dot_general(
        wegT_ref[...], e2, (((1,), (1,)), ((), ())),
        preferred_element_type=jnp.float32)
    msk = msk_ref[0]
    q = q_ref[0]
    k = k_ref[0]
    v = v_ref[0]
    outs = []
    for h in range(_H):
        sl = slice(h * _HD, (h + 1) * _HD)
        bias = jnp.concatenate(
            [rawT[h:h + 1, i * _N:(i + 1) * _N] for i in range(_BI)], axis=0)
        bias = (bias - me * sweg_ref[h]) * re
        s = jax.lax.dot_general(
            q[:, sl], k[:, sl], (((1,), (1,)), ((), ())),
            preferred_element_type=jnp.float32)
        s = s + bias
        s = jnp.where(msk == 0, _NEG, s)
        mx = jnp.max(s, axis=-1, keepdims=True)
        p = jnp.exp(s - mx)
        l = jnp.sum(p, axis=-1, keepdims=True)
        oh = jnp.dot(p, v[:, sl], preferred_element_type=jnp.float32)
        outs.append(oh / l)
    o = jnp.concatenate(outs, axis=1)
    attn = jnp.dot(o, wo_ref[...], preferred_element_type=jnp.float32) + bo_ref[...]
    t = x_ref[0] + attn
    mt = jnp.mean(t, axis=-1, keepdims=True)
    vt = jnp.mean(t * t, axis=-1, keepdims=True) - mt * mt
    tn = (t - mt) * jax.lax.rsqrt(vt + _EPS) * fg_ref[...] + fb_ref[...]
    h1 = jnp.dot(tn, w1_ref[...], preferred_element_type=jnp.float32) + b1_ref[...]
    g1 = 0.5 * h1 * (1.0 + jax.lax.erf(h1 * (2.0 ** -0.5)))
    h2 = jnp.dot(g1, w2_ref[...], preferred_element_type=jnp.float32) + b2_ref[...]
    o_ref[0] = t + h2


def kernel(node_features, edge_features, adjacency_mask, node_norm_g,
           node_norm_b, edge_norm_g, edge_norm_b, Wq, bq, Wk, bk, Wv, bv,
           Wo, bo, We, be, ff_norm_g, ff_norm_b, W1, b1, W2, b2):
    ng = node_norm_g.reshape(1, _D)
    nb = node_norm_b.reshape(1, _D)
    bq2 = bq.reshape(1, _D)
    bv2 = bv.reshape(1, _D)
    wegT = (We * edge_norm_g[:, None]).T
    sweg = jnp.sum(wegT, axis=1)
    bo2 = bo.reshape(1, _D)
    fg = ff_norm_g.reshape(1, _D)
    fb = ff_norm_b.reshape(1, _D)
    b1_2 = b1.reshape(1, _DFF)
    b2_2 = b2.reshape(1, _D)

    full = lambda shape: pl.BlockSpec(shape, lambda b: tuple(0 for _ in shape))
    q, k, v = pl.pallas_call(
        _qkv_kernel,
        grid=(_B,),
        in_specs=[
            pl.BlockSpec((1, _N, _D), lambda b: (b, 0, 0)),
            full((1, _D)), full((1, _D)),
            full((_D, _D)), full((1, _D)),
            full((_D, _D)),
            full((_D, _D)), full((1, _D)),
        ],
        out_specs=[pl.BlockSpec((1, _N, _D), lambda b: (b, 0, 0))] * 3,
        out_shape=[jax.ShapeDtypeStruct((_B, _N, _D), jnp.float32)] * 3,
        compiler_params=pltpu.CompilerParams(
            dimension_semantics=("parallel",)),
        name="gt_qkv",
    )(node_features, ng, nb, Wq, bq2, Wk, Wv, bv2)

    full2 = lambda shape: pl.BlockSpec(shape, lambda b, i: tuple(0 for _ in shape))
    out = pl.pallas_call(
        _attn_ffn_kernel,
        grid=(_B, _N // _BI),
        in_specs=[
            pl.BlockSpec((1, _BI, _N, _D), lambda b, i: (b, i, 0, 0)),
            pl.BlockSpec((1, _BI, _N), lambda b, i: (b, i, 0)),
            pl.BlockSpec((1, _BI, _D), lambda b, i: (b, i, 0)),
            pl.BlockSpec((1, _N, _D), lambda b, i: (b, 0, 0)),
            pl.BlockSpec((1, _N, _D), lambda b, i: (b, 0, 0)),
            pl.BlockSpec((1, _BI, _D), lambda b, i: (b, i, 0)),
            full2((_H, _D)),
            pl.BlockSpec(memory_space=pltpu.SMEM),
            full2((_D, _D)), full2((1, _D)),
            full2((1, _D)), full2((1, _D)),
            full2((_D, _DFF)), full2((1, _DFF)),
            full2((_DFF, _D)), full2((1, _D)),
        ],
        out_specs=pl.BlockSpec((1, _BI, _D), lambda b, i: (b, i, 0)),
        out_shape=jax.ShapeDtypeStruct((_B, _N, _D), jnp.float32),
        compiler_params=pltpu.CompilerParams(
            dimension_semantics=("parallel", "parallel"),
            vmem_limit_bytes=52 * 1024 * 1024),
        name="gt_attn_ffn",
    )(edge_features, adjacency_mask, q, k, v, node_features,
      wegT, sweg, Wo, bo2, fg, fb, W1, b1_2, W2, b2_2)
    return out


# speedup vs baseline: 3.6245x; 2.6390x over previous
import jax
import jax.numpy as jnp
from jax.experimental import pallas as pl
from jax.experimental.pallas import tpu as pltpu

_B, _N, _D, _H = 2, 512, 256, 8
_HD = _D // _H
_DFF = 4 * _D
_EPS = 1e-5
_BI = 16
_R = _BI * _H
_NEG = -1e30


def _qkv_kernel(x_ref, g_ref, b_ref, wq_ref, bq_ref, wk_ref, wv_ref, bv_ref,
                q_ref, k_ref, v_ref):
    x = x_ref[0]
    m = jnp.mean(x, axis=-1, keepdims=True)
    var = jnp.mean(x * x, axis=-1, keepdims=True) - m * m
    nn = (x - m) * jax.lax.rsqrt(var + _EPS) * g_ref[...] + b_ref[...]
    scale = 1.0 / (_HD ** 0.5)
    q_ref[0] = (jnp.dot(nn, wq_ref[...], preferred_element_type=jnp.float32)
                + bq_ref[...]) * scale
    k_ref[0] = jnp.dot(nn, wk_ref[...], preferred_element_type=jnp.float32)
    v_ref[0] = (jnp.dot(nn, wv_ref[...], preferred_element_type=jnp.float32)
                + bv_ref[...])


def _attn_ffn_kernel(e_ref, msk_ref, q_ref, k_ref, v_ref, x_ref,
                     lhs_ref, hm_ref, mexp_ref, wo_ref, bo_ref, fg_ref, fb_ref,
                     w1_ref, b1_ref, w2_ref, b2_ref, o_ref):
    e = e_ref[0]
    e2 = e.reshape(_BI * _N, _D)
    dotA = jax.lax.dot_general(
        lhs_ref[...], e2, (((1,), (1,)), ((), ())),
        preferred_element_type=jnp.float32)
    msq = jnp.mean(e * e, axis=-1)

    def gather(lo):
        return jnp.concatenate(
            [dotA[lo:lo + _H, i * _N:(i + 1) * _N] for i in range(_BI)], axis=0)
    raw_all = gather(0)
    msw_all = gather(8)
    me_all = gather(16)
    mexp = mexp_ref[...]
    msq_all = jnp.dot(mexp, msq, preferred_element_type=jnp.float32)
    ve_all = msq_all - me_all * me_all
    bias_all = (raw_all - msw_all) * jax.lax.rsqrt(ve_all + _EPS)

    mterm = jnp.where(msk_ref[0] == 0, _NEG, 0.0)
    mask_all = jnp.dot(mexp, mterm, preferred_element_type=jnp.float32)
    hm = hm_ref[...]
    q_all = jnp.dot(mexp, q_ref[0], preferred_element_type=jnp.float32) * hm
    s = jax.lax.dot_general(
        q_all, k_ref[0], (((1,), (1,)), ((), ())),
        preferred_element_type=jnp.float32) + bias_all + mask_all
    mx = jnp.max(s, axis=-1, keepdims=True)
    p = jnp.exp(s - mx)
    l = jnp.sum(p, axis=-1, keepdims=True)
    pv = jnp.dot(p, v_ref[0], preferred_element_type=jnp.float32)
    pvm = (pv / l) * hm
    o = jax.lax.dot_general(
        mexp, pvm, (((0,), (0,)), ((), ())),
        preferred_element_type=jnp.float32)

    attn = jnp.dot(o, wo_ref[...], preferred_element_type=jnp.float32) + bo_ref[...]
    t = x_ref[0] + attn
    mt = jnp.mean(t, axis=-1, keepdims=True)
    vt = jnp.mean(t * t, axis=-1, keepdims=True) - mt * mt
    tn = (t - mt) * jax.lax.rsqrt(vt + _EPS) * fg_ref[...] + fb_ref[...]
    h1 = jnp.dot(tn, w1_ref[...], preferred_element_type=jnp.float32) + b1_ref[...]
    g1 = 0.5 * h1 * (1.0 + jax.lax.erf(h1 * (2.0 ** -0.5)))
    h2 = jnp.dot(g1, w2_ref[...], preferred_element_type=jnp.float32) + b2_ref[...]
    o_ref[0] = t + h2


def kernel(node_features, edge_features, adjacency_mask, node_norm_g,
           node_norm_b, edge_norm_g, edge_norm_b, Wq, bq, Wk, bk, Wv, bv,
           Wo, bo, We, be, ff_norm_g, ff_norm_b, W1, b1, W2, b2):
    ng = node_norm_g.reshape(1, _D)
    nb = node_norm_b.reshape(1, _D)
    bq2 = bq.reshape(1, _D)
    bv2 = bv.reshape(1, _D)
    wegT = (We * edge_norm_g[:, None]).T
    sweg = jnp.sum(wegT, axis=1)
    ones = jnp.ones((_H, _D), jnp.float32)
    lhs_aug = jnp.concatenate(
        [wegT, ones * (sweg[:, None] / _D), ones / _D], axis=0)
    hm = (jnp.arange(_D, dtype=jnp.int32)[None, :] // _HD
          == jnp.arange(_H, dtype=jnp.int32)[:, None]).astype(jnp.float32)
    hm_tiled = jnp.tile(hm, (_BI, 1))
    mexp = (jnp.arange(_R, dtype=jnp.int32)[:, None] // _H
            == jnp.arange(_BI, dtype=jnp.int32)[None, :]).astype(jnp.float32)
    bo2 = bo.reshape(1, _D)
    fg = ff_norm_g.reshape(1, _D)
    fb = ff_norm_b.reshape(1, _D)
    b1_2 = b1.reshape(1, _DFF)
    b2_2 = b2.reshape(1, _D)

    full = lambda shape: pl.BlockSpec(shape, lambda b: tuple(0 for _ in shape))
    q, k, v = pl.pallas_call(
        _qkv_kernel,
        grid=(_B,),
        in_specs=[
            pl.BlockSpec((1, _N, _D), lambda b: (b, 0, 0)),
            full((1, _D)), full((1, _D)),
            full((_D, _D)), full((1, _D)),
            full((_D, _D)),
            full((_D, _D)), full((1, _D)),
        ],
        out_specs=[pl.BlockSpec((1, _N, _D), lambda b: (b, 0, 0))] * 3,
        out_shape=[jax.ShapeDtypeStruct((_B, _N, _D), jnp.float32)] * 3,
        compiler_params=pltpu.CompilerParams(
            dimension_semantics=("parallel",)),
        name="gt_qkv",
    )(node_features, ng, nb, Wq, bq2, Wk, Wv, bv2)

    full2 = lambda shape: pl.BlockSpec(shape, lambda b, i: tuple(0 for _ in shape))
    out = pl.pallas_call(
        _attn_ffn_kernel,
        grid=(_B, _N // _BI),
        in_specs=[
            pl.BlockSpec((1, _BI, _N, _D), lambda b, i: (b, i, 0, 0)),
            pl.BlockSpec((1, _BI, _N), lambda b, i: (b, i, 0)),
            pl.BlockSpec((1, _BI, _D), lambda b, i: (b, i, 0)),
            pl.BlockSpec((1, _N, _D), lambda b, i: (b, 0, 0)),
            pl.BlockSpec((1, _N, _D), lambda b, i: (b, 0, 0)),
            pl.BlockSpec((1, _BI, _D), lambda b, i: (b, i, 0)),
            full2((3 * _H, _D)),
            full2((_R, _D)),
            full2((_R, _BI)),
            full2((_D, _D)), full2((1, _D)),
            full2((1, _D)), full2((1, _D)),
            full2((_D, _DFF)), full2((1, _DFF)),
            full2((_DFF, _D)), full2((1, _D)),
        ],
        out_specs=pl.BlockSpec((1, _BI, _D), lambda b, i: (b, i, 0)),
        out_shape=jax.ShapeDtypeStruct((_B, _N, _D), jnp.float32),
        compiler_params=pltpu.CompilerParams(
            dimension_semantics=("parallel", "parallel"),
            vmem_limit_bytes=58 * 1024 * 1024),
        name="gt_attn_ffn",
    )(edge_features, adjacency_mask, q, k, v, node_features,
      lhs_aug, hm_tiled, mexp, Wo, bo2, fg, fb, W1, b1_2, W2, b2_2)
    return out


# speedup vs baseline: 3.8794x; 1.0703x over previous
import jax
import jax.numpy as jnp
from jax.experimental import pallas as pl
from jax.experimental.pallas import tpu as pltpu

_B, _N, _D, _H = 2, 512, 256, 8
_HD = _D // _H
_DFF = 4 * _D
_EPS = 1e-5
_BI = 16
_R = _BI * _H
_NEG = -1e30


def _qkv_kernel(x_ref, g_ref, b_ref, wq_ref, bq_ref, wk_ref, wv_ref, bv_ref,
                q_ref, k_ref, v_ref):
    x = x_ref[0]
    m = jnp.mean(x, axis=-1, keepdims=True)
    var = jnp.mean(x * x, axis=-1, keepdims=True) - m * m
    nn = (x - m) * jax.lax.rsqrt(var + _EPS) * g_ref[...] + b_ref[...]
    scale = 1.0 / (_HD ** 0.5)
    q_ref[0] = (jnp.dot(nn, wq_ref[...], preferred_element_type=jnp.float32)
                + bq_ref[...]) * scale
    k_ref[0] = jnp.dot(nn, wk_ref[...], preferred_element_type=jnp.float32)
    v_ref[0] = (jnp.dot(nn, wv_ref[...], preferred_element_type=jnp.float32)
                + bv_ref[...])


def _attn_kernel(e_ref, msk_ref, q_ref, k_ref, v_ref,
                 lhs_ref, hm_ref, mexp_ref, o_ref):
    e = e_ref[0]
    e2 = e.reshape(_BI * _N, _D)
    dotA = jax.lax.dot_general(
        lhs_ref[...], e2, (((1,), (1,)), ((), ())),
        preferred_element_type=jnp.float32)
    msq = jnp.mean(e * e, axis=-1)

    def gather(lo):
        return jnp.concatenate(
            [dotA[lo:lo + _H, i * _N:(i + 1) * _N] for i in range(_BI)], axis=0)
    raw_all = gather(0)
    msw_all = gather(8)
    me_all = gather(16)
    mexp = mexp_ref[...]
    msq_all = jnp.dot(mexp, msq, preferred_element_type=jnp.float32)
    ve_all = msq_all - me_all * me_all
    bias_all = (raw_all - msw_all) * jax.lax.rsqrt(ve_all + _EPS)

    mterm = jnp.where(msk_ref[0] == 0, _NEG, 0.0)
    mask_all = jnp.dot(mexp, mterm, preferred_element_type=jnp.float32)
    hm = hm_ref[...]
    q_all = jnp.dot(mexp, q_ref[0], preferred_element_type=jnp.float32) * hm
    s = jax.lax.dot_general(
        q_all, k_ref[0], (((1,), (1,)), ((), ())),
        preferred_element_type=jnp.float32) + bias_all + mask_all
    mx = jnp.max(s, axis=-1, keepdims=True)
    p = jnp.exp(s - mx)
    l = jnp.sum(p, axis=-1, keepdims=True)
    pv = jnp.dot(p, v_ref[0], preferred_element_type=jnp.float32)
    pvm = (pv / l) * hm
    o_ref[0] = jax.lax.dot_general(
        mexp, pvm, (((0,), (0,)), ((), ())),
        preferred_element_type=jnp.float32)


def _ffn_kernel(a_ref, x_ref, wo_ref, bo_ref, fg_ref, fb_ref,
                w1_ref, b1_ref, w2_ref, b2_ref, o_ref):
    attn = (jnp.dot(a_ref[0], wo_ref[...], preferred_element_type=jnp.float32)
            + bo_ref[...])
    t = x_ref[0] + attn
    mt = jnp.mean(t, axis=-1, keepdims=True)
    vt = jnp.mean(t * t, axis=-1, keepdims=True) - mt * mt
    tn = (t - mt) * jax.lax.rsqrt(vt + _EPS) * fg_ref[...] + fb_ref[...]
    h1 = jnp.dot(tn, w1_ref[...], preferred_element_type=jnp.float32) + b1_ref[...]
    g1 = 0.5 * h1 * (1.0 + jax.lax.erf(h1 * (2.0 ** -0.5)))
    h2 = jnp.dot(g1, w2_ref[...], preferred_element_type=jnp.float32) + b2_ref[...]
    o_ref[0] = t + h2


def kernel(node_features, edge_features, adjacency_mask, node_norm_g,
           node_norm_b, edge_norm_g, edge_norm_b, Wq, bq, Wk, bk, Wv, bv,
           Wo, bo, We, be, ff_norm_g, ff_norm_b, W1, b1, W2, b2):
    ng = node_norm_g.reshape(1, _D)
    nb = node_norm_b.reshape(1, _D)
    bq2 = bq.reshape(1, _D)
    bv2 = bv.reshape(1, _D)
    wegT = (We * edge_norm_g[:, None]).T
    sweg = jnp.sum(wegT, axis=1)
    ones = jnp.ones((_H, _D), jnp.float32)
    lhs_aug = jnp.concatenate(
        [wegT, ones * (sweg[:, None] / _D), ones / _D], axis=0)
    hm = (jnp.arange(_D, dtype=jnp.int32)[None, :] // _HD
          == jnp.arange(_H, dtype=jnp.int32)[:, None]).astype(jnp.float32)
    hm_tiled = jnp.tile(hm, (_BI, 1))
    mexp = (jnp.arange(_R, dtype=jnp.int32)[:, None] // _H
            == jnp.arange(_BI, dtype=jnp.int32)[None, :]).astype(jnp.float32)
    bo2 = bo.reshape(1, _D)
    fg = ff_norm_g.reshape(1, _D)
    fb = ff_norm_b.reshape(1, _D)
    b1_2 = b1.reshape(1, _DFF)
    b2_2 = b2.reshape(1, _D)

    full = lambda shape: pl.BlockSpec(shape, lambda b: tuple(0 for _ in shape))
    q, k, v = pl.pallas_call(
        _qkv_kernel,
        grid=(_B,),
        in_specs=[
            pl.BlockSpec((1, _N, _D), lambda b: (b, 0, 0)),
            full((1, _D)), full((1, _D)),
            full((_D, _D)), full((1, _D)),
            full((_D, _D)),
            full((_D, _D)), full((1, _D)),
        ],
        out_specs=[pl.BlockSpec((1, _N, _D), lambda b: (b, 0, 0))] * 3,
        out_shape=[jax.ShapeDtypeStruct((_B, _N, _D), jnp.float32)] * 3,
        compiler_params=pltpu.CompilerParams(
            dimension_semantics=("parallel",)),
        name="gt_qkv",
    )(node_features, ng, nb, Wq, bq2, Wk, Wv, bv2)

    full2 = lambda shape: pl.BlockSpec(shape, lambda b, i: tuple(0 for _ in shape))
    a = pl.pallas_call(
        _attn_kernel,
        grid=(_B, _N // _BI),
        in_specs=[
            pl.BlockSpec((1, _BI, _N, _D), lambda b, i: (b, i, 0, 0)),
            pl.BlockSpec((1, _BI, _N), lambda b, i: (b, i, 0)),
            pl.BlockSpec((1, _BI, _D), lambda b, i: (b, i, 0)),
            pl.BlockSpec((1, _N, _D), lambda b, i: (b, 0, 0)),
            pl.BlockSpec((1, _N, _D), lambda b, i: (b, 0, 0)),
            full2((3 * _H, _D)),
            full2((_R, _D)),
            full2((_R, _BI)),
        ],
        out_specs=pl.BlockSpec((1, _BI, _D), lambda b, i: (b, i, 0)),
        out_shape=jax.ShapeDtypeStruct((_B, _N, _D), jnp.float32),
        compiler_params=pltpu.CompilerParams(
            dimension_semantics=("parallel", "parallel"),
            vmem_limit_bytes=58 * 1024 * 1024),
        name="gt_attn",
    )(edge_features, adjacency_mask, q, k, v, lhs_aug, hm_tiled, mexp)

    out = pl.pallas_call(
        _ffn_kernel,
        grid=(_B,),
        in_specs=[
            pl.BlockSpec((1, _N, _D), lambda b: (b, 0, 0)),
            pl.BlockSpec((1, _N, _D), lambda b: (b, 0, 0)),
            full((_D, _D)), full((1, _D)),
            full((1, _D)), full((1, _D)),
            full((_D, _DFF)), full((1, _DFF)),
            full((_DFF, _D)), full((1, _D)),
        ],
        out_specs=pl.BlockSpec((1, _N, _D), lambda b: (b, 0, 0)),
        out_shape=jax.ShapeDtypeStruct((_B, _N, _D), jnp.float32),
        compiler_params=pltpu.CompilerParams(
            dimension_semantics=("parallel",)),
        name="gt_ffn",
    )(a, node_features, Wo, bo2, fg, fb, W1, b1_2, W2, b2_2)
    return out


# speedup vs baseline: 4.2615x; 1.0985x over previous
import jax
import jax.numpy as jnp
from jax.experimental import pallas as pl
from jax.experimental.pallas import tpu as pltpu

_B, _N, _D, _H = 2, 512, 256, 8
_HD = _D // _H
_DFF = 4 * _D
_EPS = 1e-5
_BI = 32
_R = _BI * _H
_NEG = -1e30


def _qkv_kernel(x_ref, g_ref, b_ref, wq_ref, bq_ref, wk_ref, wv_ref, bv_ref,
                q_ref, k_ref, v_ref):
    x = x_ref[0]
    m = jnp.mean(x, axis=-1, keepdims=True)
    var = jnp.mean(x * x, axis=-1, keepdims=True) - m * m
    nn = (x - m) * jax.lax.rsqrt(var + _EPS) * g_ref[...] + b_ref[...]
    scale = 1.0 / (_HD ** 0.5)
    q_ref[0] = (jnp.dot(nn, wq_ref[...], preferred_element_type=jnp.float32)
                + bq_ref[...]) * scale
    k_ref[0] = jnp.dot(nn, wk_ref[...], preferred_element_type=jnp.float32)
    v_ref[0] = (jnp.dot(nn, wv_ref[...], preferred_element_type=jnp.float32)
                + bv_ref[...])


def _attn_kernel(e_ref, msk_ref, q_ref, k_ref, v_ref,
                 lhs_ref, hm_ref, mexp_ref, o_ref):
    e = e_ref[0]
    e2 = e.reshape(_BI * _N, _D)
    dotA = jax.lax.dot_general(
        lhs_ref[...], e2, (((1,), (1,)), ((), ())),
        preferred_element_type=jnp.float32)
    msq = jnp.mean(e * e, axis=-1)

    def gather(lo):
        return jnp.concatenate(
            [dotA[lo:lo + _H, i * _N:(i + 1) * _N] for i in range(_BI)], axis=0)
    raw_all = gather(0)
    msw_all = gather(8)
    me_all = gather(16)
    mexp = mexp_ref[...]
    msq_all = jnp.dot(mexp, msq, preferred_element_type=jnp.float32)
    ve_all = msq_all - me_all * me_all
    bias_all = (raw_all - msw_all) * jax.lax.rsqrt(ve_all + _EPS)

    mterm = jnp.where(msk_ref[0] == 0, _NEG, 0.0)
    mask_all = jnp.dot(mexp, mterm, preferred_element_type=jnp.float32)
    hm = hm_ref[...]
    q_all = jnp.dot(mexp, q_ref[0], preferred_element_type=jnp.float32) * hm
    s = jax.lax.dot_general(
        q_all, k_ref[0], (((1,), (1,)), ((), ())),
        preferred_element_type=jnp.float32) + bias_all + mask_all
    mx = jnp.max(s, axis=-1, keepdims=True)
    p = jnp.exp(s - mx)
    l = jnp.sum(p, axis=-1, keepdims=True)
    pv = jnp.dot(p, v_ref[0], preferred_element_type=jnp.float32)
    pvm = (pv / l) * hm
    o_ref[0] = jax.lax.dot_general(
        mexp, pvm, (((0,), (0,)), ((), ())),
        preferred_element_type=jnp.float32)


def _ffn_kernel(a_ref, x_ref, wo_ref, bo_ref, fg_ref, fb_ref,
                w1_ref, b1_ref, w2_ref, b2_ref, o_ref):
    attn = (jnp.dot(a_ref[0], wo_ref[...], preferred_element_type=jnp.float32)
            + bo_ref[...])
    t = x_ref[0] + attn
    mt = jnp.mean(t, axis=-1, keepdims=True)
    vt = jnp.mean(t * t, axis=-1, keepdims=True) - mt * mt
    tn = (t - mt) * jax.lax.rsqrt(vt + _EPS) * fg_ref[...] + fb_ref[...]
    h1 = jnp.dot(tn, w1_ref[...], preferred_element_type=jnp.float32) + b1_ref[...]
    g1 = 0.5 * h1 * (1.0 + jax.lax.erf(h1 * (2.0 ** -0.5)))
    h2 = jnp.dot(g1, w2_ref[...], preferred_element_type=jnp.float32) + b2_ref[...]
    o_ref[0] = t + h2


def kernel(node_features, edge_features, adjacency_mask, node_norm_g,
           node_norm_b, edge_norm_g, edge_norm_b, Wq, bq, Wk, bk, Wv, bv,
           Wo, bo, We, be, ff_norm_g, ff_norm_b, W1, b1, W2, b2):
    ng = node_norm_g.reshape(1, _D)
    nb = node_norm_b.reshape(1, _D)
    bq2 = bq.reshape(1, _D)
    bv2 = bv.reshape(1, _D)
    wegT = (We * edge_norm_g[:, None]).T
    sweg = jnp.sum(wegT, axis=1)
    ones = jnp.ones((_H, _D), jnp.float32)
    lhs_aug = jnp.concatenate(
        [wegT, ones * (sweg[:, None] / _D), ones / _D], axis=0)
    hm = (jnp.arange(_D, dtype=jnp.int32)[None, :] // _HD
          == jnp.arange(_H, dtype=jnp.int32)[:, None]).astype(jnp.float32)
    hm_tiled = jnp.tile(hm, (_BI, 1))
    mexp = (jnp.arange(_R, dtype=jnp.int32)[:, None] // _H
            == jnp.arange(_BI, dtype=jnp.int32)[None, :]).astype(jnp.float32)
    bo2 = bo.reshape(1, _D)
    fg = ff_norm_g.reshape(1, _D)
    fb = ff_norm_b.reshape(1, _D)
    b1_2 = b1.reshape(1, _DFF)
    b2_2 = b2.reshape(1, _D)

    full = lambda shape: pl.BlockSpec(shape, lambda b: tuple(0 for _ in shape))
    q, k, v = pl.pallas_call(
        _qkv_kernel,
        grid=(_B,),
        in_specs=[
            pl.BlockSpec((1, _N, _D), lambda b: (b, 0, 0)),
            full((1, _D)), full((1, _D)),
            full((_D, _D)), full((1, _D)),
            full((_D, _D)),
            full((_D, _D)), full((1, _D)),
        ],
        out_specs=[pl.BlockSpec((1, _N, _D), lambda b: (b, 0, 0))] * 3,
        out_shape=[jax.ShapeDtypeStruct((_B, _N, _D), jnp.float32)] * 3,
        compiler_params=pltpu.CompilerParams(
            dimension_semantics=("parallel",)),
        name="gt_qkv",
    )(node_features, ng, nb, Wq, bq2, Wk, Wv, bv2)

    full2 = lambda shape: pl.BlockSpec(shape, lambda b, i: tuple(0 for _ in shape))
    a = pl.pallas_call(
        _attn_kernel,
        grid=(_B, _N // _BI),
        in_specs=[
            pl.BlockSpec((1, _BI, _N, _D), lambda b, i: (b, i, 0, 0)),
            pl.BlockSpec((1, _BI, _N), lambda b, i: (b, i, 0)),
            pl.BlockSpec((1, _BI, _D), lambda b, i: (b, i, 0)),
            pl.BlockSpec((1, _N, _D), lambda b, i: (b, 0, 0)),
            pl.BlockSpec((1, _N, _D), lambda b, i: (b, 0, 0)),
            full2((3 * _H, _D)),
            full2((_R, _D)),
            full2((_R, _BI)),
        ],
        out_specs=pl.BlockSpec((1, _BI, _D), lambda b, i: (b, i, 0)),
        out_shape=jax.ShapeDtypeStruct((_B, _N, _D), jnp.float32),
        compiler_params=pltpu.CompilerParams(
            dimension_semantics=("parallel", "parallel"),
            vmem_limit_bytes=58 * 1024 * 1024),
        name="gt_attn",
    )(edge_features, adjacency_mask, q, k, v, lhs_aug, hm_tiled, mexp)

    out = pl.pallas_call(
        _ffn_kernel,
        grid=(_B,),
        in_specs=[
            pl.BlockSpec((1, _N, _D), lambda b: (b, 0, 0)),
            pl.BlockSpec((1, _N, _D), lambda b: (b, 0, 0)),
            full((_D, _D)), full((1, _D)),
            full((1, _D)), full((1, _D)),
            full((_D, _DFF)), full((1, _DFF)),
            full((_DFF, _D)), full((1, _D)),
        ],
        out_specs=pl.BlockSpec((1, _N, _D), lambda b: (b, 0, 0)),
        out_shape=jax.ShapeDtypeStruct((_B, _N, _D), jnp.float32),
        compiler_params=pltpu.CompilerParams(
            dimension_semantics=("parallel",)),
        name="gt_ffn",
    )(a, node_features, Wo, bo2, fg, fb, W1, b1_2, W2, b2_2)
    return out
